# Initial kernel scaffold; baseline (speedup 1.0000x reference)
import jax
import jax.numpy as jnp
from jax.experimental import pallas as pl


def kernel(x, z_alpha, alpha_indices, edge_index, W1, b1, W2, b2, W_root, b_nn, Wg1, as1, ad1, bg1, Wg2, as2, ad2, bg2):
    raise NotImplementedError("write your pallas kernel here")


# speedup vs baseline: 1.0423x; 1.0423x over previous
import functools

import jax
import jax.numpy as jnp
from jax.experimental import pallas as pl
from jax.experimental.pallas import tpu as pltpu

N = 50000
E = 200000
EA = 25000
FH = 64
FA = 16
HID = 64

_EB = 512
_EAP = 25088


def _nnconv_msg_body(z_ref, xs_ref, w1_ref, b1_ref, w2s_ref, b2r_ref, out_ref):
    z = z_ref[...]
    v = jnp.dot(z, w1_ref[...], preferred_element_type=jnp.float32) + b1_ref[...]
    h = jnp.where(v >= 0, v, 0.01 * v)
    xs = xs_ref[...]
    t2 = jnp.dot(xs, w2s_ref[...], preferred_element_type=jnp.float32)
    acc = jnp.dot(xs, b2r_ref[...], preferred_element_type=jnp.float32)
    for j in range(HID):
        acc += h[:, j][:, None] * t2[:, j * FH:(j + 1) * FH]
    out_ref[...] = acc


def _nnconv_msg(z_pad, xs_pad, W1, b1, W2s, B2r):
    grid = (_EAP // _EB,)
    return pl.pallas_call(
        _nnconv_msg_body,
        grid=grid,
        in_specs=[
            pl.BlockSpec((_EB, FA), lambda i: (i, 0)),
            pl.BlockSpec((_EB, FH), lambda i: (i, 0)),
            pl.BlockSpec((FA, HID), lambda i: (0, 0)),
            pl.BlockSpec((1, HID), lambda i: (0, 0)),
            pl.BlockSpec((FH, HID * FH), lambda i: (0, 0)),
            pl.BlockSpec((FH, FH), lambda i: (0, 0)),
        ],
        out_specs=pl.BlockSpec((_EB, FH), lambda i: (i, 0)),
        out_shape=jax.ShapeDtypeStruct((_EAP, FH), jnp.float32),
    )(z_pad, xs_pad, W1, b1, W2s, B2r)


def _gat_dense(h, src, dst, W, a_s, a_d, b):
    n = h.shape[0]
    loop = jnp.arange(n, dtype=src.dtype)
    s = jnp.concatenate([src, loop])
    d = jnp.concatenate([dst, loop])
    hp = h @ W
    al_s = jnp.sum(hp * a_s, axis=-1)
    al_d = jnp.sum(hp * a_d, axis=-1)
    e = jax.nn.leaky_relu(al_s[s] + al_d[d], negative_slope=0.2)
    m = jax.ops.segment_max(e, d, num_segments=n)
    m = jnp.where(jnp.isfinite(m), m, 0.0)
    w = jnp.exp(e - m[d])
    den = jax.ops.segment_sum(w, d, num_segments=n)
    coef = w / (den[d] + 1e-16)
    out = jax.ops.segment_sum(coef[:, None] * hp[s], d, num_segments=n)
    return out + b


def kernel(x, z_alpha, alpha_indices, edge_index,
           W1, b1, W2, b2, W_root, b_nn,
           Wg1, as1, ad1, bg1, Wg2, as2, ad2, bg2):
    src_a = alpha_indices[0]
    dst_a = alpha_indices[1]
    W2s = W2.reshape(HID, FH, FH).transpose(1, 0, 2).reshape(FH, HID * FH)
    B2r = b2.reshape(FH, FH)

    xs = x[src_a]
    z_pad = jnp.pad(z_alpha, ((0, _EAP - EA), (0, 0)))
    xs_pad = jnp.pad(xs, ((0, _EAP - EA), (0, 0)))
    msg = _nnconv_msg(z_pad, xs_pad, W1, b1.reshape(1, HID), W2s, B2r)[:EA]

    out = jax.ops.segment_sum(msg, dst_a, num_segments=N) + x @ W_root + b_nn
    out = _gat_dense(out, edge_index[0], edge_index[1], Wg1, as1, ad1, bg1)
    out = _gat_dense(out, edge_index[0], edge_index[1], Wg2, as2, ad2, bg2)
    return out
